# Initial kernel scaffold; baseline (speedup 1.0000x reference)
import jax
import jax.numpy as jnp
from jax.experimental import pallas as pl


def kernel(indices, ivectors, ovectors):
    raise NotImplementedError("write your pallas kernel here")


# speedup vs baseline: 1.1258x; 1.1258x over previous
import functools

import jax
import jax.numpy as jnp
from jax import lax
from jax.experimental import pallas as pl
from jax.experimental.pallas import tpu as pltpu
from jax.experimental.pallas import tpu_sc as plsc

B, L, D = 4096, 50, 64
N = B * L
NC, NS = 2, 16
NW = NC * NS
PER_W = N // NW
CHUNK = 800
NCHUNK = PER_W // CHUNK

_MESH = plsc.VectorSubcoreMesh(
    core_axis_name="c", subcore_axis_name="s", num_cores=NC, num_subcores=NS
)


@functools.partial(
    pl.kernel,
    out_type=jax.ShapeDtypeStruct((2 * N, D), jnp.float32),
    mesh=_MESH,
    scratch_types=[
        pltpu.VMEM((CHUNK,), jnp.int32),
        pltpu.VMEM((CHUNK, D), jnp.float32),
        pltpu.VMEM((CHUNK, D), jnp.float32),
        pltpu.SemaphoreType.DMA,
        pltpu.SemaphoreType.DMA,
    ],
    compiler_params=pltpu.CompilerParams(use_tc_tiling_on_sc=False),
)
def _dual_gather(idx_hbm, itab_hbm, otab_hbm, out_hbm,
                 idx_v, rows_i, rows_o, sem_i, sem_o):
    wid = lax.axis_index("s") * NC + lax.axis_index("c")
    base = wid * PER_W

    def body(k, carry):
        off = base + k * CHUNK
        pltpu.sync_copy(idx_hbm.at[pl.ds(off, CHUNK)], idx_v)
        cp_i = pltpu.async_copy(itab_hbm.at[idx_v], rows_i, sem_i)
        cp_o = pltpu.async_copy(otab_hbm.at[idx_v], rows_o, sem_o)
        cp_i.wait()
        pltpu.sync_copy(rows_i, out_hbm.at[pl.ds(off, CHUNK)])
        cp_o.wait()
        pltpu.sync_copy(rows_o, out_hbm.at[pl.ds(N + off, CHUNK)])
        return carry

    lax.fori_loop(0, NCHUNK, body, 0)


@jax.jit
def kernel(indices, ivectors, ovectors):
    flat_idx = indices.reshape(N).astype(jnp.int32)
    out = _dual_gather(flat_idx, ivectors, ovectors)
    return out.reshape(2, B, L, D)


# speedup vs baseline: 1.1338x; 1.0071x over previous
import functools

import jax
import jax.numpy as jnp
from jax import lax
from jax.experimental import pallas as pl
from jax.experimental.pallas import tpu as pltpu
from jax.experimental.pallas import tpu_sc as plsc

B, L, D = 4096, 50, 64
N = B * L
NC, NS = 2, 16
NW = NC * NS
PER_W = N // NW
CHUNK = 400
NCHUNK = PER_W // CHUNK

_MESH = plsc.VectorSubcoreMesh(
    core_axis_name="c", subcore_axis_name="s", num_cores=NC, num_subcores=NS
)

_SCRATCH = (
    [pltpu.VMEM((CHUNK,), jnp.int32) for _ in range(NCHUNK)]
    + [
        pltpu.VMEM((2, CHUNK, D), jnp.float32),
        pltpu.VMEM((2, CHUNK, D), jnp.float32),
    ]
    + [pltpu.SemaphoreType.DMA for _ in range(10)]
)


@functools.partial(
    pl.kernel,
    out_type=jax.ShapeDtypeStruct((2 * N, D), jnp.float32),
    mesh=_MESH,
    scratch_types=_SCRATCH,
    compiler_params=pltpu.CompilerParams(use_tc_tiling_on_sc=False),
)
def _dual_gather(idx_hbm, itab_hbm, otab_hbm, out_hbm, *scratch):
    idx_v = scratch[:NCHUNK]
    rows_i, rows_o = scratch[NCHUNK], scratch[NCHUNK + 1]
    sem_idx = scratch[NCHUNK + 2]
    gi = scratch[NCHUNK + 3:NCHUNK + 5]
    go = scratch[NCHUNK + 5:NCHUNK + 7]
    wi = scratch[NCHUNK + 7:NCHUNK + 9]
    wo = scratch[NCHUNK + 9:NCHUNK + 11]
    wid = lax.axis_index("s") * NC + lax.axis_index("c")
    base = wid * PER_W
    row0 = wid * NCHUNK

    idx_cps = [
        pltpu.async_copy(idx_hbm.at[row0 + k], idx_v[k], sem_idx)
        for k in range(NCHUNK)
    ]
    for cp in idx_cps:
        cp.wait()

    gathers = {}
    writebacks = {}

    def start_gather(k, b):
        gathers[k] = (
            pltpu.async_copy(itab_hbm.at[idx_v[k]], rows_i.at[b], gi[b]),
            pltpu.async_copy(otab_hbm.at[idx_v[k]], rows_o.at[b], go[b]),
        )

    def start_writeback(k, b):
        off = base + k * CHUNK
        cp_i, cp_o = gathers[k]
        cp_i.wait()
        w_i = pltpu.async_copy(rows_i.at[b], out_hbm.at[pl.ds(off, CHUNK)],
                               wi[b])
        cp_o.wait()
        w_o = pltpu.async_copy(rows_o.at[b], out_hbm.at[pl.ds(N + off, CHUNK)],
                               wo[b])
        writebacks[k] = (w_i, w_o)

    def wait_writeback(k):
        w_i, w_o = writebacks[k]
        w_i.wait()
        w_o.wait()

    start_gather(0, 0)
    start_gather(1, 1)
    for k in range(NCHUNK):
        b = k % 2
        start_writeback(k, b)
        if k + 2 < NCHUNK:
            wait_writeback(k)
            start_gather(k + 2, b)
    wait_writeback(NCHUNK - 2)
    wait_writeback(NCHUNK - 1)


@jax.jit
def kernel(indices, ivectors, ovectors):
    idx2d = indices.reshape(NW * NCHUNK, CHUNK).astype(jnp.int32)
    out = _dual_gather(idx2d, ivectors, ovectors)
    return out.reshape(2, B, L, D)
